# baseline (speedup 1.0000x reference)
import functools

import jax
import jax.numpy as jnp
from jax import lax
from jax.experimental import pallas as pl
from jax.experimental.pallas import tpu as pltpu
from jax.experimental.pallas import tpu_sc as plsc

BATCH = 4
SEQ = 8192
D = 768

_NC = 2
_NS = 16
NW = _NC * _NS
ROWS_PER_W = SEQ // NW
C = 64
CHUNK = C * D
VECS = CHUNK // 16
NCHUNKS = ROWS_PER_W // C

_mesh = plsc.VectorSubcoreMesh(core_axis_name="c", subcore_axis_name="s")


@functools.partial(
    pl.kernel,
    mesh=_mesh,
    out_type=jax.ShapeDtypeStruct((BATCH * SEQ * D,), jnp.float32),
    scratch_types=[
        pltpu.VMEM((CHUNK,), jnp.float32),
        pltpu.VMEM((CHUNK,), jnp.float32),
    ],
)
def _sc_pos_add(x_hbm, pos_hbm, out_hbm, pos_v, x_v):
    wid = lax.axis_index("s") * _NC + lax.axis_index("c")
    base = wid * ROWS_PER_W * D

    def add_body(i, _):
        s = pl.ds(pl.multiple_of(i * 16, 16), 16)
        x_v[s] = x_v[s] + pos_v[s]
        return 0

    def chunk_body(ci, _):
        off = pl.multiple_of(base + ci * CHUNK, 256)
        pltpu.sync_copy(pos_hbm.at[pl.ds(off, CHUNK)], pos_v)

        def batch_body(b, _):
            xoff = pl.multiple_of(b * SEQ * D + off, 256)
            pltpu.sync_copy(x_hbm.at[pl.ds(xoff, CHUNK)], x_v)
            lax.fori_loop(0, VECS, add_body, 0, unroll=8)
            pltpu.sync_copy(x_v, out_hbm.at[pl.ds(xoff, CHUNK)])
            return 0

        lax.fori_loop(0, BATCH, batch_body, 0)
        return 0

    lax.fori_loop(0, NCHUNKS, chunk_body, 0)


def kernel(x, pos_table):
    out = _sc_pos_add(x.reshape(-1), pos_table.reshape(-1))
    return out.reshape(BATCH, SEQ, D)


# speedup vs baseline: 1.6680x; 1.6680x over previous
import functools

import jax
import jax.numpy as jnp
from jax import lax
from jax.experimental import pallas as pl
from jax.experimental.pallas import tpu as pltpu
from jax.experimental.pallas import tpu_sc as plsc

BATCH = 4
SEQ = 8192
D = 768

_NC = 2
_NS = 16
NW = _NC * _NS
ROWS_PER_W = SEQ // NW
C = 32
CHUNK = C * D
VECS = CHUNK // 16
NCHUNKS = ROWS_PER_W // C

_mesh = plsc.VectorSubcoreMesh(core_axis_name="c", subcore_axis_name="s")


@functools.partial(
    pl.kernel,
    mesh=_mesh,
    out_type=jax.ShapeDtypeStruct((BATCH * SEQ * D,), jnp.float32),
    scratch_types=[
        pltpu.VMEM((CHUNK,), jnp.float32),
        pltpu.VMEM((CHUNK,), jnp.float32),
        pltpu.VMEM((CHUNK,), jnp.float32),
        pltpu.VMEM((CHUNK,), jnp.float32),
        pltpu.SemaphoreType.DMA,
        pltpu.SemaphoreType.DMA,
        pltpu.SemaphoreType.DMA,
        pltpu.SemaphoreType.DMA,
        pltpu.SemaphoreType.DMA,
        pltpu.SemaphoreType.DMA,
    ],
)
def _sc_pos_add(x_hbm, pos_hbm, out_hbm, pos0, pos1, xb0, xb1,
                sp0, sp1, si0, si1, so0, so1):
    wid = lax.axis_index("s") * _NC + lax.axis_index("c")
    base = wid * ROWS_PER_W * D
    pos_bufs = (pos0, pos1)
    x_bufs = (xb0, xb1)
    sp = (sp0, sp1)
    si = (si0, si1)
    so = (so0, so1)

    def pos_off(ci):
        return pl.multiple_of(base + ci * CHUNK, 256)

    def x_off(ci, b):
        return pl.multiple_of(b * SEQ * D + base + ci * CHUNK, 256)

    pltpu.async_copy(pos_hbm.at[pl.ds(pos_off(0), CHUNK)], pos0, sp0)
    pltpu.async_copy(x_hbm.at[pl.ds(x_off(0, 0), CHUNK)], xb0, si0)

    def outer(cp, _):
        for cpi in range(2):
            ci = cp * 2 + cpi
            pref = pos_bufs[cpi]
            for b in range(BATCH):
                a = b % 2
                cur = x_bufs[a]
                other = x_bufs[1 - a]

                if b < BATCH - 1:
                    if b == 0 and cpi == 0:
                        @pl.when(cp > 0)
                        def _():
                            pltpu.make_async_copy(
                                other, out_hbm.at[pl.ds(0, CHUNK)], so[1 - a]
                            ).wait()
                    else:
                        pltpu.make_async_copy(
                            other, out_hbm.at[pl.ds(0, CHUNK)], so[1 - a]
                        ).wait()
                    pltpu.async_copy(
                        x_hbm.at[pl.ds(x_off(ci, b + 1), CHUNK)],
                        other, si[1 - a])
                else:
                    @pl.when(ci < NCHUNKS - 1)
                    def _():
                        pltpu.make_async_copy(
                            other, out_hbm.at[pl.ds(0, CHUNK)], so[1 - a]
                        ).wait()
                        pltpu.async_copy(
                            x_hbm.at[pl.ds(x_off(ci + 1, 0), CHUNK)],
                            other, si[1 - a])

                if b == 0:
                    pltpu.make_async_copy(
                        pos_hbm.at[pl.ds(0, CHUNK)], pref, sp[cpi]).wait()

                    @pl.when(ci < NCHUNKS - 1)
                    def _():
                        pltpu.async_copy(
                            pos_hbm.at[pl.ds(pos_off(ci + 1), CHUNK)],
                            pos_bufs[1 - cpi], sp[1 - cpi])

                pltpu.make_async_copy(
                    x_hbm.at[pl.ds(0, CHUNK)], cur, si[a]).wait()

                @plsc.parallel_loop(0, VECS, unroll=8)
                def _(i):
                    s = pl.ds(pl.multiple_of(i * 16, 16), 16)
                    cur[s] = cur[s] + pref[s]

                pltpu.async_copy(cur, out_hbm.at[pl.ds(x_off(ci, b), CHUNK)],
                                 so[a])
        return 0

    lax.fori_loop(0, NCHUNKS // 2, outer, 0)

    pltpu.make_async_copy(xb0, out_hbm.at[pl.ds(0, CHUNK)], so0).wait()
    pltpu.make_async_copy(xb1, out_hbm.at[pl.ds(0, CHUNK)], so1).wait()


def kernel(x, pos_table):
    out = _sc_pos_add(x.reshape(-1), pos_table.reshape(-1))
    return out.reshape(BATCH, SEQ, D)


# speedup vs baseline: 4.6469x; 2.7859x over previous
import functools

import jax
import jax.numpy as jnp
from jax import lax
from jax.experimental import pallas as pl
from jax.experimental.pallas import tpu as pltpu
from jax.experimental.pallas import tpu_sc as plsc

BATCH = 4
SEQ = 8192
D = 768
NLANE = D // 16

_NC = 2
_NS = 16
NW = _NC * _NS
ROWS_PER_W = SEQ // NW
C = 32
NCHUNKS = ROWS_PER_W // C

_mesh = plsc.VectorSubcoreMesh(core_axis_name="c", subcore_axis_name="s")


@functools.partial(
    pl.kernel,
    mesh=_mesh,
    out_type=jax.ShapeDtypeStruct((BATCH, SEQ, D), jnp.float32),
    scratch_types=[
        pltpu.VMEM((C, D), jnp.float32),
        pltpu.VMEM((C, D), jnp.float32),
        pltpu.VMEM((C, D), jnp.float32),
        pltpu.VMEM((C, D), jnp.float32),
        pltpu.SemaphoreType.DMA,
        pltpu.SemaphoreType.DMA,
        pltpu.SemaphoreType.DMA,
        pltpu.SemaphoreType.DMA,
        pltpu.SemaphoreType.DMA,
        pltpu.SemaphoreType.DMA,
    ],
)
def _sc_pos_add(x_hbm, pos_hbm, out_hbm, pos0, pos1, xb0, xb1,
                sp0, sp1, si0, si1, so0, so1):
    wid = lax.axis_index("s") * _NC + lax.axis_index("c")
    base = wid * ROWS_PER_W
    pos_bufs = (pos0, pos1)
    x_bufs = (xb0, xb1)
    sp = (sp0, sp1)
    si = (si0, si1)
    so = (so0, so1)

    def rows(ci):
        return pl.ds(pl.multiple_of(base + ci * C, C), C)

    pltpu.async_copy(pos_hbm.at[rows(0)], pos0, sp0)
    pltpu.async_copy(x_hbm.at[0, rows(0)], xb0, si0)

    def outer(cp, _):
        for cpi in range(2):
            ci = cp * 2 + cpi
            pref = pos_bufs[cpi]
            for b in range(BATCH):
                a = b % 2
                cur = x_bufs[a]
                other = x_bufs[1 - a]

                def _wait_out_other():
                    pltpu.make_async_copy(
                        other, out_hbm.at[0, rows(0)], so[1 - a]).wait()

                if b < BATCH - 1:
                    if b == 0 and cpi == 0:
                        pl.when(cp > 0)(_wait_out_other)
                    else:
                        _wait_out_other()
                    pltpu.async_copy(x_hbm.at[b + 1, rows(ci)],
                                     other, si[1 - a])
                else:
                    @pl.when(ci < NCHUNKS - 1)
                    def _():
                        _wait_out_other()
                        pltpu.async_copy(x_hbm.at[0, rows(ci + 1)],
                                         other, si[1 - a])

                if b == 0:
                    pltpu.make_async_copy(
                        pos_hbm.at[rows(0)], pref, sp[cpi]).wait()

                    @pl.when(ci < NCHUNKS - 1)
                    def _():
                        pltpu.async_copy(pos_hbm.at[rows(ci + 1)],
                                         pos_bufs[1 - cpi], sp[1 - cpi])

                pltpu.make_async_copy(
                    x_hbm.at[0, rows(0)], cur, si[a]).wait()

                @plsc.parallel_loop(0, C, unroll=2)
                def _(r):
                    for j in range(NLANE):
                        s = pl.ds(j * 16, 16)
                        cur[r, s] = cur[r, s] + pref[r, s]

                pltpu.async_copy(cur, out_hbm.at[b, rows(ci)], so[a])
        return 0

    lax.fori_loop(0, NCHUNKS // 2, outer, 0)

    pltpu.make_async_copy(xb0, out_hbm.at[0, rows(0)], so0).wait()
    pltpu.make_async_copy(xb1, out_hbm.at[0, rows(0)], so1).wait()


def kernel(x, pos_table):
    return _sc_pos_add(x, pos_table)


# speedup vs baseline: 4.8602x; 1.0459x over previous
import functools

import jax
import jax.numpy as jnp
from jax import lax
from jax.experimental import pallas as pl
from jax.experimental.pallas import tpu as pltpu
from jax.experimental.pallas import tpu_sc as plsc

BATCH = 4
SEQ = 8192
D = 768
NLANE = D // 16

_NC = 2
_NS = 16
NW = _NC * _NS
ROWS_PER_W = SEQ // NW
C = 32
NCHUNKS = ROWS_PER_W // C

_mesh = plsc.VectorSubcoreMesh(core_axis_name="c", subcore_axis_name="s")


@functools.partial(
    pl.kernel,
    mesh=_mesh,
    out_type=jax.ShapeDtypeStruct((BATCH, SEQ, D), jnp.float32),
    scratch_types=[
        pltpu.VMEM((C, D), jnp.float32),
        [pltpu.VMEM((C, D), jnp.float32) for _ in range(BATCH)],
        pltpu.SemaphoreType.DMA,
        [pltpu.SemaphoreType.DMA for _ in range(BATCH)],
        [pltpu.SemaphoreType.DMA for _ in range(BATCH)],
    ],
)
def _sc_pos_add(x_hbm, pos_hbm, out_hbm, pos_v, xb, sp, si, so):
    wid = lax.axis_index("s") * _NC + lax.axis_index("c")
    base = wid * ROWS_PER_W

    def rows(ci):
        return pl.ds(pl.multiple_of(base + ci * C, C), C)

    pltpu.async_copy(pos_hbm.at[rows(0)], pos_v, sp)
    pltpu.async_copy(x_hbm.at[0, rows(0)], xb[0], si[0])
    pltpu.async_copy(x_hbm.at[1, rows(0)], xb[1], si[1])

    def outer(ci, _):
        for b in range(BATCH):
            cur = xb[b]
            nxt = (b + 2) % BATCH

            def _wait_out(idx):
                pltpu.make_async_copy(
                    xb[idx], out_hbm.at[0, rows(0)], so[idx]).wait()

            if b < 2:
                @pl.when(ci > 0)
                def _():
                    _wait_out(nxt)
                pltpu.async_copy(x_hbm.at[b + 2, rows(ci)], xb[nxt], si[nxt])
            else:
                _wait_out(nxt)

                @pl.when(ci < NCHUNKS - 1)
                def _():
                    pltpu.async_copy(x_hbm.at[b - 2, rows(ci + 1)],
                                     xb[nxt], si[nxt])

            if b == 0:
                pltpu.make_async_copy(pos_hbm.at[rows(0)], pos_v, sp).wait()

            pltpu.make_async_copy(x_hbm.at[0, rows(0)], cur, si[b]).wait()

            @plsc.parallel_loop(0, C, unroll=2)
            def _(r):
                for j in range(NLANE):
                    s = pl.ds(j * 16, 16)
                    cur[r, s] = cur[r, s] + pos_v[r, s]

            if b == BATCH - 1:
                @pl.when(ci < NCHUNKS - 1)
                def _():
                    pltpu.async_copy(pos_hbm.at[rows(ci + 1)], pos_v, sp)

            pltpu.async_copy(cur, out_hbm.at[b, rows(ci)], so[b])
        return 0

    lax.fori_loop(0, NCHUNKS, outer, 0)

    pltpu.make_async_copy(xb[2], out_hbm.at[0, rows(0)], so[2]).wait()
    pltpu.make_async_copy(xb[3], out_hbm.at[0, rows(0)], so[3]).wait()


def kernel(x, pos_table):
    return _sc_pos_add(x, pos_table)


# speedup vs baseline: 4.8649x; 1.0010x over previous
import functools

import jax
import jax.numpy as jnp
from jax import lax
from jax.experimental import pallas as pl
from jax.experimental.pallas import tpu as pltpu
from jax.experimental.pallas import tpu_sc as plsc

BATCH = 4
SEQ = 8192
D = 768
NLANE = D // 16

_NC = 2
_NS = 16
NW = _NC * _NS
ROWS_PER_W = SEQ // NW
C = 32
NCHUNKS = ROWS_PER_W // C

_mesh = plsc.VectorSubcoreMesh(core_axis_name="c", subcore_axis_name="s")


@functools.partial(
    pl.kernel,
    mesh=_mesh,
    out_type=jax.ShapeDtypeStruct((BATCH, SEQ, D), jnp.float32),
    compiler_params=pltpu.CompilerParams(
        disable_bounds_checks=True,
        disable_semaphore_checks=True,
    ),
    scratch_types=[
        pltpu.VMEM((C, D), jnp.float32),
        [pltpu.VMEM((C, D), jnp.float32) for _ in range(BATCH)],
        pltpu.SemaphoreType.DMA,
        [pltpu.SemaphoreType.DMA for _ in range(BATCH)],
        [pltpu.SemaphoreType.DMA for _ in range(BATCH)],
    ],
)
def _sc_pos_add(x_hbm, pos_hbm, out_hbm, pos_v, xb, sp, si, so):
    wid = lax.axis_index("s") * _NC + lax.axis_index("c")
    base = wid * ROWS_PER_W

    def rows(ci):
        return pl.ds(pl.multiple_of(base + ci * C, C), C)

    pltpu.async_copy(pos_hbm.at[rows(0)], pos_v, sp)
    pltpu.async_copy(x_hbm.at[0, rows(0)], xb[0], si[0])
    pltpu.async_copy(x_hbm.at[1, rows(0)], xb[1], si[1])

    def outer(ci, _):
        for b in range(BATCH):
            cur = xb[b]
            nxt = (b + 2) % BATCH

            def _wait_out(idx):
                pltpu.make_async_copy(
                    xb[idx], out_hbm.at[0, rows(0)], so[idx]).wait()

            if b < 2:
                @pl.when(ci > 0)
                def _():
                    _wait_out(nxt)
                pltpu.async_copy(x_hbm.at[b + 2, rows(ci)], xb[nxt], si[nxt])
            else:
                _wait_out(nxt)

                @pl.when(ci < NCHUNKS - 1)
                def _():
                    pltpu.async_copy(x_hbm.at[b - 2, rows(ci + 1)],
                                     xb[nxt], si[nxt])

            if b == 0:
                pltpu.make_async_copy(pos_hbm.at[rows(0)], pos_v, sp).wait()

            pltpu.make_async_copy(x_hbm.at[0, rows(0)], cur, si[b]).wait()

            @plsc.parallel_loop(0, C, unroll=2)
            def _(r):
                for j in range(NLANE):
                    s = pl.ds(j * 16, 16)
                    cur[r, s] = cur[r, s] + pos_v[r, s]

            if b == BATCH - 1:
                @pl.when(ci < NCHUNKS - 1)
                def _():
                    pltpu.async_copy(pos_hbm.at[rows(ci + 1)], pos_v, sp)

            pltpu.async_copy(cur, out_hbm.at[b, rows(ci)], so[b])
        return 0

    lax.fori_loop(0, NCHUNKS, outer, 0)

    pltpu.make_async_copy(xb[2], out_hbm.at[0, rows(0)], so[2]).wait()
    pltpu.make_async_copy(xb[3], out_hbm.at[0, rows(0)], so[3]).wait()


def kernel(x, pos_table):
    return _sc_pos_add(x, pos_table)


# speedup vs baseline: 5.4093x; 1.1119x over previous
import functools

import jax
import jax.numpy as jnp
from jax import lax
from jax.experimental import pallas as pl
from jax.experimental.pallas import tpu as pltpu
from jax.experimental.pallas import tpu_sc as plsc

BATCH = 4
SEQ = 8192
D = 768
NLANE = D // 16

_NC = 2
_NS = 16
NW = _NC * _NS
ROWS_PER_W = SEQ // NW
C = 16
NCHUNKS = ROWS_PER_W // C

_mesh = plsc.VectorSubcoreMesh(core_axis_name="c", subcore_axis_name="s")


@functools.partial(
    pl.kernel,
    mesh=_mesh,
    out_type=jax.ShapeDtypeStruct((BATCH, SEQ, D), jnp.float32),
    compiler_params=pltpu.CompilerParams(
        disable_bounds_checks=True,
        disable_semaphore_checks=True,
    ),
    scratch_types=[
        [pltpu.VMEM((C, D), jnp.float32) for _ in range(2)],
        [[pltpu.VMEM((C, D), jnp.float32) for _ in range(BATCH)]
         for _ in range(2)],
        [pltpu.SemaphoreType.DMA for _ in range(2)],
        [[pltpu.SemaphoreType.DMA for _ in range(BATCH)]
         for _ in range(2)],
        [[pltpu.SemaphoreType.DMA for _ in range(BATCH)]
         for _ in range(2)],
    ],
)
def _sc_pos_add(x_hbm, pos_hbm, out_hbm, pos_v, xb, sp, si, so):
    wid = lax.axis_index("s") * _NC + lax.axis_index("c")
    base = wid * ROWS_PER_W

    def rows(ci):
        return pl.ds(pl.multiple_of(base + ci * C, C), C)

    pltpu.async_copy(pos_hbm.at[rows(0)], pos_v[0], sp[0])
    for b in range(BATCH):
        pltpu.async_copy(x_hbm.at[b, rows(0)], xb[0][b], si[0][b])

    def outer(ci2, _):
        for ph in range(2):
            ci = ci2 * 2 + ph
            pos = pos_v[ph]

            def _drain_prev_outs():
                for b in range(BATCH):
                    pltpu.make_async_copy(
                        xb[1 - ph][b], out_hbm.at[0, rows(0)],
                        so[1 - ph][b]).wait()

            if ph == 0:
                pl.when(ci2 > 0)(_drain_prev_outs)
            else:
                _drain_prev_outs()

            @pl.when(ci < NCHUNKS - 1)
            def _():
                for b in range(BATCH):
                    pltpu.async_copy(x_hbm.at[b, rows(ci + 1)],
                                     xb[1 - ph][b], si[1 - ph][b])
                pltpu.async_copy(pos_hbm.at[rows(ci + 1)],
                                 pos_v[1 - ph], sp[1 - ph])

            pltpu.make_async_copy(pos_hbm.at[rows(0)], pos, sp[ph]).wait()
            for b in range(BATCH):
                pltpu.make_async_copy(
                    x_hbm.at[0, rows(0)], xb[ph][b], si[ph][b]).wait()

            @plsc.parallel_loop(0, C, unroll=2)
            def _(r):
                for j in range(NLANE):
                    s = pl.ds(j * 16, 16)
                    p = pos[r, s]
                    vals = [xb[ph][b][r, s] + p for b in range(BATCH)]
                    for b in range(BATCH):
                        xb[ph][b][r, s] = vals[b]

            for b in range(BATCH):
                pltpu.async_copy(xb[ph][b], out_hbm.at[b, rows(ci)],
                                 so[ph][b])
        return 0

    lax.fori_loop(0, NCHUNKS // 2, outer, 0)

    for b in range(BATCH):
        pltpu.make_async_copy(xb[1][b], out_hbm.at[0, rows(0)],
                              so[1][b]).wait()


def kernel(x, pos_table):
    return _sc_pos_add(x, pos_table)
